# Initial kernel scaffold; baseline (speedup 1.0000x reference)
import jax
import jax.numpy as jnp
from jax.experimental import pallas as pl


def kernel(points, W1, b1, W2, b2, gamma, beta):
    raise NotImplementedError("write your pallas kernel here")


# speedup vs baseline: 1.1822x; 1.1822x over previous
import functools

import jax
import jax.numpy as jnp
from jax.experimental import pallas as pl
from jax.experimental.pallas import tpu as pltpu

X0, Y0 = -50.0, -50.0
CELL = 0.5
H = 200
W = 200
K = H * W
EMBED = 96
B = 4
N = 131072
BK = B * K
BKPAD = 160128
CORES = 2
ROWS_BLK = 4000


def _mlp_ln_body(acc_ref, w1_ref, b1_ref, w2_ref, b2_ref, g_ref, bt_ref, out_ref):
    a = acc_ref[0] + acc_ref[1]
    cnt = a[:, 4:5]
    denom = jnp.maximum(cnt, 1.0)
    mean = a[:, :4] / denom
    feat8 = jnp.concatenate([mean, cnt, jnp.zeros_like(a[:, :3])], axis=1)
    h = jnp.maximum(
        jnp.dot(feat8, w1_ref[...], preferred_element_type=jnp.float32) + b1_ref[...], 0.0)
    tok = jnp.dot(h, w2_ref[...], preferred_element_type=jnp.float32) + b2_ref[...]
    mu = jnp.mean(tok, axis=-1, keepdims=True)
    var = jnp.mean((tok - mu) ** 2, axis=-1, keepdims=True)
    out_ref[...] = (tok - mu) * jax.lax.rsqrt(var + 1e-5) * g_ref[...] + bt_ref[...]


def _mlp_ln(acc, W1p, b1, W2, b2, gamma, beta):
    grid = (BK // ROWS_BLK,)
    return pl.pallas_call(
        _mlp_ln_body,
        grid=grid,
        in_specs=[
            pl.BlockSpec((CORES, ROWS_BLK, 8), lambda i: (0, i, 0)),
            pl.BlockSpec((8, 64), lambda i: (0, 0)),
            pl.BlockSpec((1, 64), lambda i: (0, 0)),
            pl.BlockSpec((64, EMBED), lambda i: (0, 0)),
            pl.BlockSpec((1, EMBED), lambda i: (0, 0)),
            pl.BlockSpec((1, EMBED), lambda i: (0, 0)),
            pl.BlockSpec((1, EMBED), lambda i: (0, 0)),
        ],
        out_specs=pl.BlockSpec((ROWS_BLK, EMBED), lambda i: (i, 0)),
        out_shape=jax.ShapeDtypeStruct((BK, EMBED), jnp.float32),
    )(acc, W1p, b1, W2, b2, gamma, beta)


def kernel(points, W1, b1, W2, b2, gamma, beta):
    xs = (points[..., 0] - X0) / CELL
    ys = (points[..., 1] - Y0) / CELL
    xi = jnp.floor(xs).astype(jnp.int32)
    yi = jnp.floor(ys).astype(jnp.int32)
    valid = (xi >= 0) & (xi < W) & (yi >= 0) & (yi < H)
    flat = jnp.clip(yi, 0, H - 1) * W + jnp.clip(xi, 0, W - 1)
    boff = jnp.arange(B, dtype=jnp.int32)[:, None] * K
    idx = jnp.where(valid, flat + boff, BK).reshape(-1)
    vm = valid[..., None].astype(jnp.float32)
    rows = jnp.concatenate(
        [points * vm, vm, jnp.zeros((B, N, 3), jnp.float32)], axis=-1).reshape(-1, 8)
    acc0 = jnp.zeros((BKPAD, 8), jnp.float32).at[idx].add(rows)
    acc = jnp.stack([acc0, jnp.zeros((BKPAD, 8), jnp.float32)])

    W1p = jnp.zeros((8, 64), jnp.float32).at[:5].set(W1)
    out = _mlp_ln(acc, W1p, b1.reshape(1, 64), W2, b2.reshape(1, EMBED),
                  gamma.reshape(1, EMBED), beta.reshape(1, EMBED))
    return out.reshape(B, K, EMBED)


# speedup vs baseline: 1.8292x; 1.5473x over previous
import functools

import jax
import jax.numpy as jnp
from jax import lax
from jax.experimental import pallas as pl
from jax.experimental.pallas import tpu as pltpu
from jax.experimental.pallas import tpu_sc as plsc

X0, Y0 = -50.0, -50.0
CELL = 0.5
H = 200
W = 200
K = H * W
EMBED = 96
B = 4
N = 131072
BK = B * K
CORES = 2
SUBS = 16
BPC = B // CORES
ACC = BPC * K
ACCPAD = 80128
PPT = (BPC * N) // SUBS
CHUNK = 2048
NCHUNK = PPT // CHUNK
GRP = 128
NGRP = CHUNK // GRP
RPT = ACCPAD // SUBS
ZCH = RPT // 2
ROWS_BLK = 4000


def _sc_body(pts_hbm, out_hbm, pts_v, rows_v, idx_v, zeros_v, acc_sh):
    c = lax.axis_index("c")
    s = lax.axis_index("s")
    lane = jax.lax.iota(jnp.int32, 16)
    zf = jnp.zeros((16,), jnp.float32)
    cc = [jnp.full((16,), i, jnp.int32) for i in range(8)]

    def zfill(i, carry):
        f = i * 16 + lane
        plsc.store_scatter(zeros_v, [lax.shift_right_logical(f, 3),
                                     lax.bitwise_and(f, 7)], zf)
        return carry
    lax.fori_loop(0, ZCH * 8 // 16, zfill, 0)
    for t in range(2):
        pltpu.sync_copy(zeros_v, acc_sh.at[pl.ds(s * RPT + t * ZCH, ZCH)])

    def ifill(i, carry):
        pids = i * 16 + lane
        plsc.store_scatter(rows_v, [pids, cc[4]], zf + 1.0)
        plsc.store_scatter(rows_v, [pids, cc[5]], zf)
        plsc.store_scatter(rows_v, [pids, cc[6]], zf)
        plsc.store_scatter(rows_v, [pids, cc[7]], zf)
        return carry
    lax.fori_loop(0, CHUNK // 16, ifill, 0)
    plsc.subcore_barrier()

    base = c * (BPC * N) + s * PPT
    boff = (s // (SUBS // BPC)) * K
    for ci in range(NCHUNK):
        pltpu.sync_copy(pts_hbm.at[pl.ds(base + ci * CHUNK, CHUNK)], pts_v)

        def pbody(j, carry):
            pids = j * 16 + lane
            x = plsc.load_gather(pts_v, [pids, cc[0]])
            y = plsc.load_gather(pts_v, [pids, cc[1]])
            z = plsc.load_gather(pts_v, [pids, cc[2]])
            w = plsc.load_gather(pts_v, [pids, cc[3]])
            xs = x * 2.0 + 100.0
            ys = y * 2.0 + 100.0
            xi = xs.astype(jnp.int32)
            yi = ys.astype(jnp.int32)
            valid = ((xs >= 0.0) & (xi < W) & (ys >= 0.0) & (yi < H))
            bin_ = jnp.where(valid, yi * W + xi + boff, ACC)
            plsc.store_scatter(rows_v, [pids, cc[0]], x)
            plsc.store_scatter(rows_v, [pids, cc[1]], y)
            plsc.store_scatter(rows_v, [pids, cc[2]], z)
            plsc.store_scatter(rows_v, [pids, cc[3]], w)
            flat = j * 16
            plsc.store_scatter(
                idx_v, [lax.shift_right_logical(flat + lane, 7),
                        lax.bitwise_and(flat + lane, 127)], bin_)
            return carry
        lax.fori_loop(0, CHUNK // 16, pbody, 0)

        for g in range(NGRP):
            pltpu.sync_copy(rows_v.at[pl.ds(g * GRP, GRP)],
                            acc_sh.at[idx_v.at[g]], add=True)

    plsc.subcore_barrier()

    for t in range(2):
        sl = pl.ds(s * RPT + t * ZCH, ZCH)
        pltpu.sync_copy(acc_sh.at[sl], out_hbm.at[c, sl])


def _sc_scatter(points_flat):
    mesh = plsc.VectorSubcoreMesh(core_axis_name="c", subcore_axis_name="s",
                                  num_cores=CORES, num_subcores=SUBS)
    return pl.kernel(
        _sc_body,
        out_type=jax.ShapeDtypeStruct((CORES, ACCPAD, 8), jnp.float32),
        mesh=mesh,
        scratch_types=[
            pltpu.VMEM((CHUNK, 4), jnp.float32),
            pltpu.VMEM((CHUNK, 8), jnp.float32),
            pltpu.VMEM((NGRP, GRP), jnp.int32),
            pltpu.VMEM((ZCH, 8), jnp.float32),
            pltpu.VMEM_SHARED((ACCPAD, 8), jnp.float32),
        ],
        compiler_params=pltpu.CompilerParams(needs_layout_passes=False,
                                             use_tc_tiling_on_sc=False),
    )(points_flat)


def _mlp_ln_body(acc_ref, w1_ref, b1_ref, w2_ref, b2_ref, g_ref, bt_ref, out_ref):
    a = acc_ref[0]
    cnt = a[:, 4:5]
    denom = jnp.maximum(cnt, 1.0)
    mean = a[:, :4] / denom
    feat8 = jnp.concatenate([mean, cnt, jnp.zeros_like(a[:, :3])], axis=1)
    h = jnp.maximum(
        jnp.dot(feat8, w1_ref[...], preferred_element_type=jnp.float32) + b1_ref[...], 0.0)
    tok = jnp.dot(h, w2_ref[...], preferred_element_type=jnp.float32) + b2_ref[...]
    mu = jnp.mean(tok, axis=-1, keepdims=True)
    var = jnp.mean((tok - mu) ** 2, axis=-1, keepdims=True)
    out_ref[...] = (tok - mu) * jax.lax.rsqrt(var + 1e-5) * g_ref[...] + bt_ref[...]


def _mlp_ln(acc, W1p, b1, W2, b2, gamma, beta):
    blocks_per_core = ACC // ROWS_BLK
    grid = (CORES, blocks_per_core)
    return pl.pallas_call(
        _mlp_ln_body,
        grid=grid,
        in_specs=[
            pl.BlockSpec((1, ROWS_BLK, 8), lambda c, j: (c, j, 0)),
            pl.BlockSpec((8, 64), lambda c, j: (0, 0)),
            pl.BlockSpec((1, 64), lambda c, j: (0, 0)),
            pl.BlockSpec((64, EMBED), lambda c, j: (0, 0)),
            pl.BlockSpec((1, EMBED), lambda c, j: (0, 0)),
            pl.BlockSpec((1, EMBED), lambda c, j: (0, 0)),
            pl.BlockSpec((1, EMBED), lambda c, j: (0, 0)),
        ],
        out_specs=pl.BlockSpec(
            (ROWS_BLK, EMBED),
            lambda c, j: (c * (ACC // ROWS_BLK) + j, 0)),
        out_shape=jax.ShapeDtypeStruct((BK, EMBED), jnp.float32),
    )(acc, W1p, b1, W2, b2, gamma, beta)


def kernel(points, W1, b1, W2, b2, gamma, beta):
    acc = _sc_scatter(points.reshape(B * N, 4))
    W1p = jnp.zeros((8, 64), jnp.float32).at[:5].set(W1)
    out = _mlp_ln(acc, W1p, b1.reshape(1, 64), W2, b2.reshape(1, EMBED),
                  gamma.reshape(1, EMBED), beta.reshape(1, EMBED))
    return out.reshape(B, K, EMBED)


# speedup vs baseline: 2.0825x; 1.1385x over previous
import functools

import jax
import jax.numpy as jnp
from jax import lax
from jax.experimental import pallas as pl
from jax.experimental.pallas import tpu as pltpu
from jax.experimental.pallas import tpu_sc as plsc

X0, Y0 = -50.0, -50.0
CELL = 0.5
H = 200
W = 200
K = H * W
EMBED = 96
B = 4
N = 131072
BK = B * K
CORES = 2
SUBS = 16
BPC = B // CORES
ACC = BPC * K
ACCPAD = 80128
PPT = (BPC * N) // SUBS
CHUNK = 2048
NCHUNK = PPT // CHUNK
GRP = 128
NGRP = CHUNK // GRP
RPT = ACCPAD // SUBS
ZCH = RPT // 2
ROWS_BLK = 4000


def _sc_body(pts_hbm, out_hbm, pts_v, rows_v, idx_v, zeros_v, acc_sh):
    c = lax.axis_index("c")
    s = lax.axis_index("s")
    lane = jax.lax.iota(jnp.int32, 16)
    zf = jnp.zeros((16,), jnp.float32)
    cc = [jnp.full((16,), i, jnp.int32) for i in range(8)]

    def zfill(i, carry):
        f = i * 16 + lane
        plsc.store_scatter(zeros_v, [lax.shift_right_logical(f, 3),
                                     lax.bitwise_and(f, 7)], zf)
        return carry
    lax.fori_loop(0, ZCH * 8 // 16, zfill, 0)
    for t in range(2):
        pltpu.sync_copy(zeros_v, acc_sh.at[pl.ds(s * RPT + t * ZCH, ZCH)])

    def ifill(i, carry):
        pids = i * 16 + lane
        plsc.store_scatter(rows_v, [pids, cc[4]], zf + 1.0)
        plsc.store_scatter(rows_v, [pids, cc[5]], zf)
        plsc.store_scatter(rows_v, [pids, cc[6]], zf)
        plsc.store_scatter(rows_v, [pids, cc[7]], zf)
        return carry
    lax.fori_loop(0, CHUNK // 16, ifill, 0)
    plsc.subcore_barrier()

    rbase = (c * (BPC * N) + s * PPT) * 4 // 128
    boff = (s // (SUBS // BPC)) * K
    for ci in range(NCHUNK):
        pltpu.sync_copy(pts_hbm.at[pl.ds(rbase + ci * (CHUNK * 4 // 128),
                                         CHUNK * 4 // 128)], pts_v)

        def pbody(j, carry):
            pids = j * 16 + lane
            prow = lax.shift_right_logical(pids, 5)
            pcol = lax.bitwise_and(pids * 4, 127)
            x = plsc.load_gather(pts_v, [prow, pcol])
            y = plsc.load_gather(pts_v, [prow, pcol + 1])
            z = plsc.load_gather(pts_v, [prow, pcol + 2])
            w = plsc.load_gather(pts_v, [prow, pcol + 3])
            xs = x * 2.0 + 100.0
            ys = y * 2.0 + 100.0
            xi = xs.astype(jnp.int32)
            yi = ys.astype(jnp.int32)
            valid = ((xs >= 0.0) & (xi < W) & (ys >= 0.0) & (yi < H))
            bin_ = jnp.where(valid, yi * W + xi + boff, ACC)
            plsc.store_scatter(rows_v, [pids, cc[0]], x)
            plsc.store_scatter(rows_v, [pids, cc[1]], y)
            plsc.store_scatter(rows_v, [pids, cc[2]], z)
            plsc.store_scatter(rows_v, [pids, cc[3]], w)
            flat = j * 16
            plsc.store_scatter(
                idx_v, [lax.shift_right_logical(flat + lane, 7),
                        lax.bitwise_and(flat + lane, 127)], bin_)
            return carry
        lax.fori_loop(0, CHUNK // 16, pbody, 0)

        for g in range(NGRP):
            pltpu.sync_copy(rows_v.at[pl.ds(g * GRP, GRP)],
                            acc_sh.at[idx_v.at[g]], add=True)

    plsc.subcore_barrier()

    for t in range(2):
        sl = pl.ds(s * RPT + t * ZCH, ZCH)
        pltpu.sync_copy(acc_sh.at[sl], out_hbm.at[c, sl])


def _sc_scatter(points_flat):
    mesh = plsc.VectorSubcoreMesh(core_axis_name="c", subcore_axis_name="s",
                                  num_cores=CORES, num_subcores=SUBS)
    return pl.kernel(
        _sc_body,
        out_type=jax.ShapeDtypeStruct((CORES, ACCPAD, 8), jnp.float32),
        mesh=mesh,
        scratch_types=[
            pltpu.VMEM((CHUNK * 4 // 128, 128), jnp.float32),
            pltpu.VMEM((CHUNK, 8), jnp.float32),
            pltpu.VMEM((NGRP, GRP), jnp.int32),
            pltpu.VMEM((ZCH, 8), jnp.float32),
            pltpu.VMEM_SHARED((ACCPAD, 8), jnp.float32),
        ],
        compiler_params=pltpu.CompilerParams(needs_layout_passes=False,
                                             use_tc_tiling_on_sc=False),
    )(points_flat)


def _mlp_ln_body(acc_ref, w1_ref, b1_ref, w2_ref, b2_ref, g_ref, bt_ref, out_ref):
    a = acc_ref[0]
    cnt = a[:, 4:5]
    denom = jnp.maximum(cnt, 1.0)
    mean = a[:, :4] / denom
    feat8 = jnp.concatenate([mean, cnt, jnp.zeros_like(a[:, :3])], axis=1)
    h = jnp.maximum(
        jnp.dot(feat8, w1_ref[...], preferred_element_type=jnp.float32) + b1_ref[...], 0.0)
    tok = jnp.dot(h, w2_ref[...], preferred_element_type=jnp.float32) + b2_ref[...]
    mu = jnp.mean(tok, axis=-1, keepdims=True)
    var = jnp.mean((tok - mu) ** 2, axis=-1, keepdims=True)
    out_ref[...] = (tok - mu) * jax.lax.rsqrt(var + 1e-5) * g_ref[...] + bt_ref[...]


def _mlp_ln(acc, W1p, b1, W2, b2, gamma, beta):
    blocks_per_core = ACC // ROWS_BLK
    grid = (CORES, blocks_per_core)
    return pl.pallas_call(
        _mlp_ln_body,
        grid=grid,
        in_specs=[
            pl.BlockSpec((1, ROWS_BLK, 8), lambda c, j: (c, j, 0)),
            pl.BlockSpec((8, 64), lambda c, j: (0, 0)),
            pl.BlockSpec((1, 64), lambda c, j: (0, 0)),
            pl.BlockSpec((64, EMBED), lambda c, j: (0, 0)),
            pl.BlockSpec((1, EMBED), lambda c, j: (0, 0)),
            pl.BlockSpec((1, EMBED), lambda c, j: (0, 0)),
            pl.BlockSpec((1, EMBED), lambda c, j: (0, 0)),
        ],
        out_specs=pl.BlockSpec(
            (ROWS_BLK, EMBED),
            lambda c, j: (c * (ACC // ROWS_BLK) + j, 0)),
        out_shape=jax.ShapeDtypeStruct((BK, EMBED), jnp.float32),
    )(acc, W1p, b1, W2, b2, gamma, beta)


def kernel(points, W1, b1, W2, b2, gamma, beta):
    acc = _sc_scatter(points.reshape(B * N * 4 // 128, 128))
    W1p = jnp.zeros((8, 64), jnp.float32).at[:5].set(W1)
    out = _mlp_ln(acc, W1p, b1.reshape(1, 64), W2, b2.reshape(1, EMBED),
                  gamma.reshape(1, EMBED), beta.reshape(1, EMBED))
    return out.reshape(B, K, EMBED)


# speedup vs baseline: 2.2299x; 1.0708x over previous
import functools

import jax
import jax.numpy as jnp
from jax import lax
from jax.experimental import pallas as pl
from jax.experimental.pallas import tpu as pltpu
from jax.experimental.pallas import tpu_sc as plsc

X0, Y0 = -50.0, -50.0
CELL = 0.5
H = 200
W = 200
K = H * W
EMBED = 96
B = 4
N = 131072
BK = B * K
CORES = 2
SUBS = 16
BPC = B // CORES
ACC = BPC * K
ACCPAD = 80128
PPT = (BPC * N) // SUBS
CHUNK = 2048
NCHUNK = PPT // CHUNK
GRP = 128
NGRP = CHUNK // GRP
RPT = ACCPAD // SUBS
ZCH = RPT // 2
ROWS_BLK = 2048


def _sc_body(pts_hbm, out_hbm, pts_v, rows_v, idx_v, zeros_v, acc_sh):
    c = lax.axis_index("c")
    s = lax.axis_index("s")
    lane = jax.lax.iota(jnp.int32, 16)
    zf = jnp.zeros((16,), jnp.float32)
    cc = [jnp.full((16,), i, jnp.int32) for i in range(8)]

    def zfill(i, carry):
        f = i * 16 + lane
        plsc.store_scatter(zeros_v, [lax.shift_right_logical(f, 3),
                                     lax.bitwise_and(f, 7)], zf)
        return carry
    lax.fori_loop(0, ZCH * 8 // 16, zfill, 0)
    for t in range(2):
        pltpu.sync_copy(zeros_v, acc_sh.at[pl.ds(s * RPT + t * ZCH, ZCH)])

    def ifill(i, carry):
        pids = i * 16 + lane
        plsc.store_scatter(rows_v, [pids, cc[4]], zf + 1.0)
        plsc.store_scatter(rows_v, [pids, cc[5]], zf)
        plsc.store_scatter(rows_v, [pids, cc[6]], zf)
        plsc.store_scatter(rows_v, [pids, cc[7]], zf)
        return carry
    lax.fori_loop(0, CHUNK // 16, ifill, 0)
    plsc.subcore_barrier()

    rbase = (c * (BPC * N) + s * PPT) * 4 // 128
    boff = (s // (SUBS // BPC)) * K
    for ci in range(NCHUNK):
        pltpu.sync_copy(pts_hbm.at[pl.ds(rbase + ci * (CHUNK * 4 // 128),
                                         CHUNK * 4 // 128)], pts_v)

        def pbody(j, carry):
            pids = j * 16 + lane
            prow = lax.shift_right_logical(pids, 5)
            pcol = lax.bitwise_and(pids * 4, 127)
            x = plsc.load_gather(pts_v, [prow, pcol])
            y = plsc.load_gather(pts_v, [prow, pcol + 1])
            z = plsc.load_gather(pts_v, [prow, pcol + 2])
            w = plsc.load_gather(pts_v, [prow, pcol + 3])
            xs = x * 2.0 + 100.0
            ys = y * 2.0 + 100.0
            xi = xs.astype(jnp.int32)
            yi = ys.astype(jnp.int32)
            valid = ((xs >= 0.0) & (xi < W) & (ys >= 0.0) & (yi < H))
            bin_ = jnp.where(valid, yi * W + xi + boff, ACC)
            plsc.store_scatter(rows_v, [pids, cc[0]], x)
            plsc.store_scatter(rows_v, [pids, cc[1]], y)
            plsc.store_scatter(rows_v, [pids, cc[2]], z)
            plsc.store_scatter(rows_v, [pids, cc[3]], w)
            flat = j * 16
            plsc.store_scatter(
                idx_v, [lax.shift_right_logical(flat + lane, 7),
                        lax.bitwise_and(flat + lane, 127)], bin_)
            return carry
        lax.fori_loop(0, CHUNK // 16, pbody, 0)

        for g in range(NGRP):
            pltpu.sync_copy(rows_v.at[pl.ds(g * GRP, GRP)],
                            acc_sh.at[idx_v.at[g]], add=True)

    plsc.subcore_barrier()

    for t in range(2):
        sl = pl.ds(s * RPT + t * ZCH, ZCH)
        pltpu.sync_copy(acc_sh.at[sl], out_hbm.at[c, sl])


def _sc_scatter(points_flat):
    mesh = plsc.VectorSubcoreMesh(core_axis_name="c", subcore_axis_name="s",
                                  num_cores=CORES, num_subcores=SUBS)
    return pl.kernel(
        _sc_body,
        out_type=jax.ShapeDtypeStruct((CORES, ACCPAD, 8), jnp.float32),
        mesh=mesh,
        scratch_types=[
            pltpu.VMEM((CHUNK * 4 // 128, 128), jnp.float32),
            pltpu.VMEM((CHUNK, 8), jnp.float32),
            pltpu.VMEM((NGRP, GRP), jnp.int32),
            pltpu.VMEM((ZCH, 8), jnp.float32),
            pltpu.VMEM_SHARED((ACCPAD, 8), jnp.float32),
        ],
        compiler_params=pltpu.CompilerParams(needs_layout_passes=False,
                                             use_tc_tiling_on_sc=False),
    )(points_flat)


_CHUNKS = [(o, min(ROWS_BLK, K - o)) for o in range(0, K, ROWS_BLK)]


def _mlp_ln_body(acc_hbm, w1t_ref, b1_ref, w2t_ref, b2_ref, g_ref, bt_ref,
                 out_ref, abuf, sem):
    b = pl.program_id(0)
    core = b // 2
    rowbase = (b % 2) * K

    def start(ci):
        off, sz = _CHUNKS[ci]
        pltpu.make_async_copy(
            acc_hbm.at[core, pl.ds(rowbase + off, sz), :],
            abuf.at[ci % 2, pl.ds(0, sz), :], sem.at[ci % 2]).start()

    def wait(ci):
        off, sz = _CHUNKS[ci]
        pltpu.make_async_copy(
            acc_hbm.at[core, pl.ds(rowbase + off, sz), :],
            abuf.at[ci % 2, pl.ds(0, sz), :], sem.at[ci % 2]).wait()

    start(0)
    for ci in range(len(_CHUNKS)):
        off, sz = _CHUNKS[ci]
        if ci + 1 < len(_CHUNKS):
            start(ci + 1)
        wait(ci)
        a = abuf[ci % 2, pl.ds(0, sz), :]
        denom = jnp.maximum(a[:, 4:5], 1.0)
        col = jax.lax.broadcasted_iota(jnp.int32, (sz, 8), 1)
        feat = jnp.where(col == 4, a, a / denom)
        h_t = jnp.maximum(
            jax.lax.dot_general(w1t_ref[...], feat, (((1,), (1,)), ((), ())),
                                preferred_element_type=jnp.float32) + b1_ref[...],
            0.0)
        tok_t = jnp.dot(w2t_ref[...], h_t,
                        preferred_element_type=jnp.float32) + b2_ref[...]
        mu = jnp.mean(tok_t, axis=0, keepdims=True)
        var = jnp.mean((tok_t - mu) ** 2, axis=0, keepdims=True)
        out_ref[0, :, pl.ds(off, sz)] = (
            (tok_t - mu) * jax.lax.rsqrt(var + 1e-5) * g_ref[...] + bt_ref[...])


def _mlp_ln(acc, W1t, b1, W2t, b2, gamma, beta):
    grid = (B,)
    return pl.pallas_call(
        _mlp_ln_body,
        grid=grid,
        in_specs=[
            pl.BlockSpec(memory_space=pl.ANY),
            pl.BlockSpec((64, 8), lambda b: (0, 0)),
            pl.BlockSpec((64, 1), lambda b: (0, 0)),
            pl.BlockSpec((EMBED, 64), lambda b: (0, 0)),
            pl.BlockSpec((EMBED, 1), lambda b: (0, 0)),
            pl.BlockSpec((EMBED, 1), lambda b: (0, 0)),
            pl.BlockSpec((EMBED, 1), lambda b: (0, 0)),
        ],
        out_specs=pl.BlockSpec((1, EMBED, K), lambda b: (b, 0, 0)),
        out_shape=jax.ShapeDtypeStruct((B, EMBED, K), jnp.float32),
        scratch_shapes=[
            pltpu.VMEM((2, ROWS_BLK, 8), jnp.float32),
            pltpu.SemaphoreType.DMA((2,)),
        ],
    )(acc, W1t, b1, W2t, b2, gamma, beta)


def kernel(points, W1, b1, W2, b2, gamma, beta):
    acc = _sc_scatter(points.reshape(B * N * 4 // 128, 128))
    W1t = jnp.zeros((64, 8), jnp.float32).at[:, :5].set(W1.T)
    out = _mlp_ln(acc, W1t, b1.reshape(64, 1), W2.T, b2.reshape(EMBED, 1),
                  gamma.reshape(EMBED, 1), beta.reshape(EMBED, 1))
    return out.swapaxes(1, 2)


# speedup vs baseline: 10.1093x; 4.5336x over previous
import functools

import jax
import jax.numpy as jnp
from jax import lax
from jax.experimental import pallas as pl
from jax.experimental.pallas import tpu as pltpu
from jax.experimental.pallas import tpu_sc as plsc

X0, Y0 = -50.0, -50.0
CELL = 0.5
H = 200
W = 200
K = H * W
EMBED = 96
B = 4
N = 131072
BK = B * K
CORES = 2
SUBS = 16
BPC = B // CORES
ACC = BPC * K
ACCPAD = 80128
PPT = (BPC * N) // SUBS
CHUNK = 2048
NCHUNK = PPT // CHUNK
GRP = 128
NGRP = CHUNK // GRP
RPT = ACCPAD // SUBS
ZCH = RPT // 2
ROWS_BLK = 2048


def _sc_body(pts_hbm, out_hbm, pts_v, rows_v, idx_v, zeros_v, acc_sh):
    c = lax.axis_index("c")
    s = lax.axis_index("s")
    lane = jax.lax.iota(jnp.int32, 16)
    zf = jnp.zeros((16,), jnp.float32)
    cc = [jnp.full((16,), i, jnp.int32) for i in range(8)]

    def zfill(i, carry):
        f = i * 16 + lane
        plsc.store_scatter(zeros_v, [lax.shift_right_logical(f, 3),
                                     lax.bitwise_and(f, 7)], zf)
        return carry
    lax.fori_loop(0, ZCH * 8 // 16, zfill, 0)
    for t in range(2):
        pltpu.sync_copy(zeros_v, acc_sh.at[pl.ds(s * RPT + t * ZCH, ZCH)])

    def ifill(i, carry):
        pids = i * 16 + lane
        plsc.store_scatter(rows_v, [pids, cc[4]], zf + 1.0)
        plsc.store_scatter(rows_v, [pids, cc[5]], zf)
        plsc.store_scatter(rows_v, [pids, cc[6]], zf)
        plsc.store_scatter(rows_v, [pids, cc[7]], zf)
        return carry
    lax.fori_loop(0, CHUNK // 16, ifill, 0)
    plsc.subcore_barrier()

    b = c * BPC + s // (SUBS // BPC)
    pblk = (s % (SUBS // BPC)) * (PPT // 128)
    boff = (s // (SUBS // BPC)) * K
    for ci in range(NCHUNK):
        rbase = b * (N // 128 * 4) + (pblk + ci * (CHUNK // 128)) * 4
        pltpu.sync_copy(pts_hbm.at[pl.ds(rbase, CHUNK * 4 // 128)], pts_v)

        def pbody(j, carry):
            pids = j * 16 + lane
            rx = jnp.zeros((16,), jnp.int32) + lax.shift_right_logical(j, 3) * 4
            pcol = lax.bitwise_and(j, 7) * 16 + lane
            x = plsc.load_gather(pts_v, [rx, pcol])
            y = plsc.load_gather(pts_v, [rx + 1, pcol])
            z = plsc.load_gather(pts_v, [rx + 2, pcol])
            w = plsc.load_gather(pts_v, [rx + 3, pcol])
            xs = x * 2.0 + 100.0
            ys = y * 2.0 + 100.0
            xi = xs.astype(jnp.int32)
            yi = ys.astype(jnp.int32)
            valid = ((xs >= 0.0) & (xi < W) & (ys >= 0.0) & (yi < H))
            bin_ = jnp.where(valid, yi * W + xi + boff, ACC)
            plsc.store_scatter(rows_v, [pids, cc[0]], x)
            plsc.store_scatter(rows_v, [pids, cc[1]], y)
            plsc.store_scatter(rows_v, [pids, cc[2]], z)
            plsc.store_scatter(rows_v, [pids, cc[3]], w)
            flat = j * 16
            plsc.store_scatter(
                idx_v, [lax.shift_right_logical(flat + lane, 7),
                        lax.bitwise_and(flat + lane, 127)], bin_)
            return carry
        lax.fori_loop(0, CHUNK // 16, pbody, 0)

        for g in range(NGRP):
            pltpu.sync_copy(rows_v.at[pl.ds(g * GRP, GRP)],
                            acc_sh.at[idx_v.at[g]], add=True)

    plsc.subcore_barrier()

    for t in range(2):
        sl = pl.ds(s * RPT + t * ZCH, ZCH)
        pltpu.sync_copy(acc_sh.at[sl], out_hbm.at[c, sl])


def _sc_scatter(points_flat):
    mesh = plsc.VectorSubcoreMesh(core_axis_name="c", subcore_axis_name="s",
                                  num_cores=CORES, num_subcores=SUBS)
    return pl.kernel(
        _sc_body,
        out_type=jax.ShapeDtypeStruct((CORES, ACCPAD, 8), jnp.float32),
        mesh=mesh,
        scratch_types=[
            pltpu.VMEM((CHUNK * 4 // 128, 128), jnp.float32),
            pltpu.VMEM((CHUNK, 8), jnp.float32),
            pltpu.VMEM((NGRP, GRP), jnp.int32),
            pltpu.VMEM((ZCH, 8), jnp.float32),
            pltpu.VMEM_SHARED((ACCPAD, 8), jnp.float32),
        ],
        compiler_params=pltpu.CompilerParams(needs_layout_passes=False,
                                             use_tc_tiling_on_sc=False),
    )(points_flat)


_CHUNKS = [(o, min(ROWS_BLK, K - o)) for o in range(0, K, ROWS_BLK)]


def _mlp_ln_body(acc_hbm, w1t_ref, b1_ref, w2t_ref, b2_ref, g_ref, bt_ref,
                 out_ref, abuf, sem):
    b = pl.program_id(0)
    core = b // 2
    rowbase = (b % 2) * K

    def start(ci):
        off, sz = _CHUNKS[ci]
        pltpu.make_async_copy(
            acc_hbm.at[core, pl.ds(rowbase + off, sz), :],
            abuf.at[ci % 2, pl.ds(0, sz), :], sem.at[ci % 2]).start()

    def wait(ci):
        off, sz = _CHUNKS[ci]
        pltpu.make_async_copy(
            acc_hbm.at[core, pl.ds(rowbase + off, sz), :],
            abuf.at[ci % 2, pl.ds(0, sz), :], sem.at[ci % 2]).wait()

    start(0)
    for ci in range(len(_CHUNKS)):
        off, sz = _CHUNKS[ci]
        if ci + 1 < len(_CHUNKS):
            start(ci + 1)
        wait(ci)
        a = abuf[ci % 2, pl.ds(0, sz), :]
        denom = jnp.maximum(a[:, 4:5], 1.0)
        col = jax.lax.broadcasted_iota(jnp.int32, (sz, 8), 1)
        feat = jnp.where(col == 4, a, a / denom)
        h_t = jnp.maximum(
            jax.lax.dot_general(w1t_ref[...], feat, (((1,), (1,)), ((), ())),
                                preferred_element_type=jnp.float32) + b1_ref[...],
            0.0)
        tok_t = jnp.dot(w2t_ref[...], h_t,
                        preferred_element_type=jnp.float32) + b2_ref[...]
        mu = jnp.mean(tok_t, axis=0, keepdims=True)
        var = jnp.mean((tok_t - mu) ** 2, axis=0, keepdims=True)
        out_ref[0, :, pl.ds(off, sz)] = (
            (tok_t - mu) * jax.lax.rsqrt(var + 1e-5) * g_ref[...] + bt_ref[...])


def _mlp_ln(acc, W1t, b1, W2t, b2, gamma, beta):
    grid = (B,)
    return pl.pallas_call(
        _mlp_ln_body,
        grid=grid,
        in_specs=[
            pl.BlockSpec(memory_space=pl.ANY),
            pl.BlockSpec((64, 8), lambda b: (0, 0)),
            pl.BlockSpec((64, 1), lambda b: (0, 0)),
            pl.BlockSpec((EMBED, 64), lambda b: (0, 0)),
            pl.BlockSpec((EMBED, 1), lambda b: (0, 0)),
            pl.BlockSpec((EMBED, 1), lambda b: (0, 0)),
            pl.BlockSpec((EMBED, 1), lambda b: (0, 0)),
        ],
        out_specs=pl.BlockSpec((1, EMBED, K), lambda b: (b, 0, 0)),
        out_shape=jax.ShapeDtypeStruct((B, EMBED, K), jnp.float32),
        scratch_shapes=[
            pltpu.VMEM((2, ROWS_BLK, 8), jnp.float32),
            pltpu.SemaphoreType.DMA((2,)),
        ],
    )(acc, W1t, b1, W2t, b2, gamma, beta)


def kernel(points, W1, b1, W2, b2, gamma, beta):
    pts_lin = (points.reshape(B, N // 128, 128, 4)
               .swapaxes(2, 3)
               .reshape(B * N * 4 // 128, 128))
    acc = _sc_scatter(pts_lin)
    W1t = jnp.zeros((64, 8), jnp.float32).at[:, :5].set(W1.T)
    out = _mlp_ln(acc, W1t, b1.reshape(64, 1), W2.T, b2.reshape(EMBED, 1),
                  gamma.reshape(EMBED, 1), beta.reshape(EMBED, 1))
    return out.swapaxes(1, 2)


# speedup vs baseline: 10.7036x; 1.0588x over previous
import functools

import jax
import jax.numpy as jnp
from jax import lax
from jax.experimental import pallas as pl
from jax.experimental.pallas import tpu as pltpu
from jax.experimental.pallas import tpu_sc as plsc

X0, Y0 = -50.0, -50.0
CELL = 0.5
H = 200
W = 200
K = H * W
EMBED = 96
B = 4
N = 131072
BK = B * K
CORES = 2
SUBS = 16
BPC = B // CORES
ACC = BPC * K
ACCPAD = 80128
PPT = (BPC * N) // SUBS
CHUNK = 2048
NCHUNK = PPT // CHUNK
GRP = 128
NGRP = CHUNK // GRP
RPT = ACCPAD // SUBS
ZCH = RPT // 2
ROWS_BLK = 4096


def _sc_body(pts_hbm, out_hbm, pts_v, rows_v, idx_v, zeros_v, acc_sh):
    c = lax.axis_index("c")
    s = lax.axis_index("s")
    lane = jax.lax.iota(jnp.int32, 16)
    zf = jnp.zeros((16,), jnp.float32)
    cc = [jnp.full((16,), i, jnp.int32) for i in range(8)]

    def zfill(i, carry):
        f = i * 16 + lane
        plsc.store_scatter(zeros_v, [lax.shift_right_logical(f, 3),
                                     lax.bitwise_and(f, 7)], zf)
        return carry
    lax.fori_loop(0, ZCH * 8 // 16, zfill, 0)
    for t in range(2):
        pltpu.sync_copy(zeros_v, acc_sh.at[pl.ds(s * RPT + t * ZCH, ZCH)])

    def ifill(i, carry):
        pids = i * 16 + lane
        plsc.store_scatter(rows_v, [pids, cc[4]], zf + 1.0)
        plsc.store_scatter(rows_v, [pids, cc[5]], zf)
        plsc.store_scatter(rows_v, [pids, cc[6]], zf)
        plsc.store_scatter(rows_v, [pids, cc[7]], zf)
        return carry
    lax.fori_loop(0, CHUNK // 16, ifill, 0)
    plsc.subcore_barrier()

    b = c * BPC + s // (SUBS // BPC)
    pblk = (s % (SUBS // BPC)) * (PPT // 128)
    boff = (s // (SUBS // BPC)) * K
    for ci in range(NCHUNK):
        rbase = b * (N // 128 * 4) + (pblk + ci * (CHUNK // 128)) * 4
        pltpu.sync_copy(pts_hbm.at[pl.ds(rbase, CHUNK * 4 // 128)], pts_v)

        def pbody(j, carry):
            pids = j * 16 + lane
            rx = jnp.zeros((16,), jnp.int32) + lax.shift_right_logical(j, 3) * 4
            pcol = lax.bitwise_and(j, 7) * 16 + lane
            x = plsc.load_gather(pts_v, [rx, pcol])
            y = plsc.load_gather(pts_v, [rx + 1, pcol])
            z = plsc.load_gather(pts_v, [rx + 2, pcol])
            w = plsc.load_gather(pts_v, [rx + 3, pcol])
            xs = x * 2.0 + 100.0
            ys = y * 2.0 + 100.0
            xi = xs.astype(jnp.int32)
            yi = ys.astype(jnp.int32)
            valid = ((xs >= 0.0) & (xi < W) & (ys >= 0.0) & (yi < H))
            bin_ = jnp.where(valid, yi * W + xi + boff, ACC)
            plsc.store_scatter(rows_v, [pids, cc[0]], x)
            plsc.store_scatter(rows_v, [pids, cc[1]], y)
            plsc.store_scatter(rows_v, [pids, cc[2]], z)
            plsc.store_scatter(rows_v, [pids, cc[3]], w)
            flat = j * 16
            plsc.store_scatter(
                idx_v, [lax.shift_right_logical(flat + lane, 7),
                        lax.bitwise_and(flat + lane, 127)], bin_)
            return carry
        lax.fori_loop(0, CHUNK // 16, pbody, 0)

        for g in range(NGRP):
            pltpu.sync_copy(rows_v.at[pl.ds(g * GRP, GRP)],
                            acc_sh.at[idx_v.at[g]], add=True)

    plsc.subcore_barrier()

    for t in range(2):
        sl = pl.ds(s * RPT + t * ZCH, ZCH)
        pltpu.sync_copy(acc_sh.at[sl], out_hbm.at[c, sl])


def _sc_scatter(points_flat):
    mesh = plsc.VectorSubcoreMesh(core_axis_name="c", subcore_axis_name="s",
                                  num_cores=CORES, num_subcores=SUBS)
    return pl.kernel(
        _sc_body,
        out_type=jax.ShapeDtypeStruct((CORES, ACCPAD, 8), jnp.float32),
        mesh=mesh,
        scratch_types=[
            pltpu.VMEM((CHUNK * 4 // 128, 128), jnp.float32),
            pltpu.VMEM((CHUNK, 8), jnp.float32),
            pltpu.VMEM((NGRP, GRP), jnp.int32),
            pltpu.VMEM((ZCH, 8), jnp.float32),
            pltpu.VMEM_SHARED((ACCPAD, 8), jnp.float32),
        ],
        compiler_params=pltpu.CompilerParams(needs_layout_passes=False,
                                             use_tc_tiling_on_sc=False),
    )(points_flat)


_CHUNKS = [(o, min(ROWS_BLK, K - o)) for o in range(0, K, ROWS_BLK)]


def _mlp_ln_body(acc_hbm, w1t_ref, b1_ref, w2t_ref, b2_ref, g_ref, bt_ref,
                 out_ref, abuf, sem):
    b = pl.program_id(0)
    core = b // 2
    rowbase = (b % 2) * K

    def start(ci):
        off, sz = _CHUNKS[ci]
        pltpu.make_async_copy(
            acc_hbm.at[core, pl.ds(rowbase + off, sz), :],
            abuf.at[ci % 2, pl.ds(0, sz), :], sem.at[ci % 2]).start()

    def wait(ci):
        off, sz = _CHUNKS[ci]
        pltpu.make_async_copy(
            acc_hbm.at[core, pl.ds(rowbase + off, sz), :],
            abuf.at[ci % 2, pl.ds(0, sz), :], sem.at[ci % 2]).wait()

    start(0)
    for ci in range(len(_CHUNKS)):
        off, sz = _CHUNKS[ci]
        if ci + 1 < len(_CHUNKS):
            start(ci + 1)
        wait(ci)
        a = abuf[ci % 2, pl.ds(0, sz), :]
        denom = jnp.maximum(a[:, 4:5], 1.0)
        col = jax.lax.broadcasted_iota(jnp.int32, (sz, 8), 1)
        feat = jnp.where(col == 4, a, a / denom)
        h_t = jnp.maximum(
            jax.lax.dot_general(w1t_ref[...], feat, (((1,), (1,)), ((), ())),
                                preferred_element_type=jnp.float32) + b1_ref[...],
            0.0)
        tok_t = jnp.dot(w2t_ref[...], h_t,
                        preferred_element_type=jnp.float32) + b2_ref[...]
        mu = jnp.mean(tok_t, axis=0, keepdims=True)
        var = jnp.mean((tok_t - mu) ** 2, axis=0, keepdims=True)
        out_ref[0, :, pl.ds(off, sz)] = (
            (tok_t - mu) * jax.lax.rsqrt(var + 1e-5) * g_ref[...] + bt_ref[...])


def _mlp_ln(acc, W1t, b1, W2t, b2, gamma, beta):
    grid = (B,)
    return pl.pallas_call(
        _mlp_ln_body,
        grid=grid,
        in_specs=[
            pl.BlockSpec(memory_space=pl.ANY),
            pl.BlockSpec((64, 8), lambda b: (0, 0)),
            pl.BlockSpec((64, 1), lambda b: (0, 0)),
            pl.BlockSpec((EMBED, 64), lambda b: (0, 0)),
            pl.BlockSpec((EMBED, 1), lambda b: (0, 0)),
            pl.BlockSpec((EMBED, 1), lambda b: (0, 0)),
            pl.BlockSpec((EMBED, 1), lambda b: (0, 0)),
        ],
        out_specs=pl.BlockSpec((1, EMBED, K), lambda b: (b, 0, 0)),
        out_shape=jax.ShapeDtypeStruct((B, EMBED, K), jnp.float32),
        scratch_shapes=[
            pltpu.VMEM((2, ROWS_BLK, 8), jnp.float32),
            pltpu.SemaphoreType.DMA((2,)),
        ],
    )(acc, W1t, b1, W2t, b2, gamma, beta)


def kernel(points, W1, b1, W2, b2, gamma, beta):
    pts_lin = (points.reshape(B, N // 128, 128, 4)
               .swapaxes(2, 3)
               .reshape(B * N * 4 // 128, 128))
    acc = _sc_scatter(pts_lin)
    W1t = jnp.zeros((64, 8), jnp.float32).at[:, :5].set(W1.T)
    out = _mlp_ln(acc, W1t, b1.reshape(64, 1), W2.T, b2.reshape(EMBED, 1),
                  gamma.reshape(EMBED, 1), beta.reshape(EMBED, 1))
    return out.swapaxes(1, 2)


# speedup vs baseline: 11.3510x; 1.0605x over previous
import functools

import jax
import jax.numpy as jnp
from jax import lax
from jax.experimental import pallas as pl
from jax.experimental.pallas import tpu as pltpu
from jax.experimental.pallas import tpu_sc as plsc

X0, Y0 = -50.0, -50.0
CELL = 0.5
H = 200
W = 200
K = H * W
EMBED = 96
B = 4
N = 131072
BK = B * K
CORES = 2
SUBS = 16
BPC = B // CORES
ACC = BPC * K
ACCPAD = 80128
PPT = (BPC * N) // SUBS
CHUNK = 2048
NCHUNK = PPT // CHUNK
GRP = 128
NGRP = CHUNK // GRP
RPT = ACCPAD // SUBS
ZCH = RPT // 2
ROWS_BLK = 4096


def _sc_body(pts_hbm, out_hbm, pts_v, rows_v, idx_v, zeros_v, acc_sh, sem):
    c = lax.axis_index("c")
    s = lax.axis_index("s")
    lane = jax.lax.iota(jnp.int32, 16)
    zf = jnp.zeros((16,), jnp.float32)
    cc = [jnp.full((16,), i, jnp.int32) for i in range(8)]

    def zfill(i, carry):
        f = i * 16 + lane
        plsc.store_scatter(zeros_v, [lax.shift_right_logical(f, 3),
                                     lax.bitwise_and(f, 7)], zf)
        return carry
    lax.fori_loop(0, ZCH * 8 // 16, zfill, 0)
    for t in range(2):
        pltpu.sync_copy(zeros_v, acc_sh.at[pl.ds(s * RPT + t * ZCH, ZCH)])

    def ifill(i, carry):
        pids = i * 16 + lane
        for sl in range(2):
            plsc.store_scatter(rows_v.at[sl], [pids, cc[4]], zf + 1.0)
            plsc.store_scatter(rows_v.at[sl], [pids, cc[5]], zf)
            plsc.store_scatter(rows_v.at[sl], [pids, cc[6]], zf)
            plsc.store_scatter(rows_v.at[sl], [pids, cc[7]], zf)
        return carry
    lax.fori_loop(0, CHUNK // 16, ifill, 0)
    plsc.subcore_barrier()

    b = c * BPC + s // (SUBS // BPC)
    pblk = (s % (SUBS // BPC)) * (PPT // 128)
    boff = (s // (SUBS // BPC)) * K
    pending = {0: [], 1: []}
    for ci in range(NCHUNK):
        sl = ci % 2
        for d in pending[sl]:
            d.wait()
        pending[sl] = []
        rbase = b * (N // 128 * 4) + (pblk + ci * (CHUNK // 128)) * 4
        pltpu.sync_copy(pts_hbm.at[pl.ds(rbase, CHUNK * 4 // 128)], pts_v)

        def pbody(j, carry):
            pids = j * 16 + lane
            rx = jnp.zeros((16,), jnp.int32) + lax.shift_right_logical(j, 3) * 4
            pcol = lax.bitwise_and(j, 7) * 16 + lane
            x = plsc.load_gather(pts_v, [rx, pcol])
            y = plsc.load_gather(pts_v, [rx + 1, pcol])
            z = plsc.load_gather(pts_v, [rx + 2, pcol])
            w = plsc.load_gather(pts_v, [rx + 3, pcol])
            xs = x * 2.0 + 100.0
            ys = y * 2.0 + 100.0
            xi = xs.astype(jnp.int32)
            yi = ys.astype(jnp.int32)
            valid = ((xs >= 0.0) & (xi < W) & (ys >= 0.0) & (yi < H))
            bin_ = jnp.where(valid, yi * W + xi + boff, ACC)
            plsc.store_scatter(rows_v.at[sl], [pids, cc[0]], x)
            plsc.store_scatter(rows_v.at[sl], [pids, cc[1]], y)
            plsc.store_scatter(rows_v.at[sl], [pids, cc[2]], z)
            plsc.store_scatter(rows_v.at[sl], [pids, cc[3]], w)
            flat = j * 16
            plsc.store_scatter(
                idx_v.at[sl], [lax.shift_right_logical(flat + lane, 7),
                               lax.bitwise_and(flat + lane, 127)], bin_)
            return carry
        lax.fori_loop(0, CHUNK // 16, pbody, 0)

        for g in range(NGRP):
            pending[sl].append(pltpu.async_copy(
                rows_v.at[sl, pl.ds(g * GRP, GRP), :],
                acc_sh.at[idx_v.at[sl, g]], sem, add=True))

    for sl in range(2):
        for d in pending[sl]:
            d.wait()
    plsc.subcore_barrier()

    for t in range(2):
        sl = pl.ds(s * RPT + t * ZCH, ZCH)
        pltpu.sync_copy(acc_sh.at[sl], out_hbm.at[c, sl])


def _sc_scatter(points_flat):
    mesh = plsc.VectorSubcoreMesh(core_axis_name="c", subcore_axis_name="s",
                                  num_cores=CORES, num_subcores=SUBS)
    return pl.kernel(
        _sc_body,
        out_type=jax.ShapeDtypeStruct((CORES, ACCPAD, 8), jnp.float32),
        mesh=mesh,
        scratch_types=[
            pltpu.VMEM((CHUNK * 4 // 128, 128), jnp.float32),
            pltpu.VMEM((2, CHUNK, 8), jnp.float32),
            pltpu.VMEM((2, NGRP, GRP), jnp.int32),
            pltpu.VMEM((ZCH, 8), jnp.float32),
            pltpu.VMEM_SHARED((ACCPAD, 8), jnp.float32),
            pltpu.SemaphoreType.DMA,
        ],
        compiler_params=pltpu.CompilerParams(needs_layout_passes=False,
                                             use_tc_tiling_on_sc=False),
    )(points_flat)


_CHUNKS = [(o, min(ROWS_BLK, K - o)) for o in range(0, K, ROWS_BLK)]


def _mlp_ln_body(acc_hbm, w1t_ref, b1_ref, w2t_ref, b2_ref, g_ref, bt_ref,
                 out_ref, abuf, sem):
    b = pl.program_id(0)
    core = b // 2
    rowbase = (b % 2) * K

    def start(ci):
        off, sz = _CHUNKS[ci]
        pltpu.make_async_copy(
            acc_hbm.at[core, pl.ds(rowbase + off, sz), :],
            abuf.at[ci % 2, pl.ds(0, sz), :], sem.at[ci % 2]).start()

    def wait(ci):
        off, sz = _CHUNKS[ci]
        pltpu.make_async_copy(
            acc_hbm.at[core, pl.ds(rowbase + off, sz), :],
            abuf.at[ci % 2, pl.ds(0, sz), :], sem.at[ci % 2]).wait()

    start(0)
    for ci in range(len(_CHUNKS)):
        off, sz = _CHUNKS[ci]
        if ci + 1 < len(_CHUNKS):
            start(ci + 1)
        wait(ci)
        a = abuf[ci % 2, pl.ds(0, sz), :]
        denom = jnp.maximum(a[:, 4:5], 1.0)
        col = jax.lax.broadcasted_iota(jnp.int32, (sz, 8), 1)
        feat = jnp.where(col == 4, a, a / denom)
        h_t = jnp.maximum(
            jax.lax.dot_general(w1t_ref[...], feat, (((1,), (1,)), ((), ())),
                                preferred_element_type=jnp.float32) + b1_ref[...],
            0.0)
        tok_t = jnp.dot(w2t_ref[...], h_t,
                        preferred_element_type=jnp.float32) + b2_ref[...]
        mu = jnp.mean(tok_t, axis=0, keepdims=True)
        var = jnp.mean((tok_t - mu) ** 2, axis=0, keepdims=True)
        out_ref[0, :, pl.ds(off, sz)] = (
            (tok_t - mu) * jax.lax.rsqrt(var + 1e-5) * g_ref[...] + bt_ref[...])


def _mlp_ln(acc, W1t, b1, W2t, b2, gamma, beta):
    grid = (B,)
    return pl.pallas_call(
        _mlp_ln_body,
        grid=grid,
        in_specs=[
            pl.BlockSpec(memory_space=pl.ANY),
            pl.BlockSpec((64, 8), lambda b: (0, 0)),
            pl.BlockSpec((64, 1), lambda b: (0, 0)),
            pl.BlockSpec((EMBED, 64), lambda b: (0, 0)),
            pl.BlockSpec((EMBED, 1), lambda b: (0, 0)),
            pl.BlockSpec((EMBED, 1), lambda b: (0, 0)),
            pl.BlockSpec((EMBED, 1), lambda b: (0, 0)),
        ],
        out_specs=pl.BlockSpec((1, EMBED, K), lambda b: (b, 0, 0)),
        out_shape=jax.ShapeDtypeStruct((B, EMBED, K), jnp.float32),
        scratch_shapes=[
            pltpu.VMEM((2, ROWS_BLK, 8), jnp.float32),
            pltpu.SemaphoreType.DMA((2,)),
        ],
    )(acc, W1t, b1, W2t, b2, gamma, beta)


def kernel(points, W1, b1, W2, b2, gamma, beta):
    pts_lin = (points.reshape(B, N // 128, 128, 4)
               .swapaxes(2, 3)
               .reshape(B * N * 4 // 128, 128))
    acc = _sc_scatter(pts_lin)
    W1t = jnp.zeros((64, 8), jnp.float32).at[:, :5].set(W1.T)
    out = _mlp_ln(acc, W1t, b1.reshape(64, 1), W2.T, b2.reshape(EMBED, 1),
                  gamma.reshape(EMBED, 1), beta.reshape(EMBED, 1))
    return out.swapaxes(1, 2)


# speedup vs baseline: 11.4917x; 1.0124x over previous
import functools

import jax
import jax.numpy as jnp
from jax import lax
from jax.experimental import pallas as pl
from jax.experimental.pallas import tpu as pltpu
from jax.experimental.pallas import tpu_sc as plsc

X0, Y0 = -50.0, -50.0
CELL = 0.5
H = 200
W = 200
K = H * W
EMBED = 96
B = 4
N = 131072
BK = B * K
CORES = 2
SUBS = 16
BPC = B // CORES
ACC = BPC * K
ACCPAD = 80128
PPT = (BPC * N) // SUBS
CHUNK = 2048
NCHUNK = PPT // CHUNK
GRP = 128
NGRP = CHUNK // GRP
RPT = ACCPAD // SUBS
ZCH = RPT // 2
ROWS_BLK = 8192


def _sc_body(pts_hbm, out_hbm, pts_v, rows_v, idx_v, zeros_v, acc_sh, sem):
    c = lax.axis_index("c")
    s = lax.axis_index("s")
    lane = jax.lax.iota(jnp.int32, 16)
    zf = jnp.zeros((16,), jnp.float32)
    cc = [jnp.full((16,), i, jnp.int32) for i in range(8)]

    def zfill(i, carry):
        f = i * 16 + lane
        plsc.store_scatter(zeros_v, [lax.shift_right_logical(f, 3),
                                     lax.bitwise_and(f, 7)], zf)
        return carry
    lax.fori_loop(0, ZCH * 8 // 16, zfill, 0)
    for t in range(2):
        pltpu.sync_copy(zeros_v, acc_sh.at[pl.ds(s * RPT + t * ZCH, ZCH)])

    def ifill(i, carry):
        pids = i * 16 + lane
        for sl in range(2):
            plsc.store_scatter(rows_v.at[sl], [pids, cc[4]], zf + 1.0)
            plsc.store_scatter(rows_v.at[sl], [pids, cc[5]], zf)
            plsc.store_scatter(rows_v.at[sl], [pids, cc[6]], zf)
            plsc.store_scatter(rows_v.at[sl], [pids, cc[7]], zf)
        return carry
    lax.fori_loop(0, CHUNK // 16, ifill, 0)
    plsc.subcore_barrier()

    b = c * BPC + s // (SUBS // BPC)
    pblk = (s % (SUBS // BPC)) * (PPT // 128)
    boff = (s // (SUBS // BPC)) * K
    pending = {0: [], 1: []}
    for ci in range(NCHUNK):
        sl = ci % 2
        for d in pending[sl]:
            d.wait()
        pending[sl] = []
        rbase = b * (N // 128 * 4) + (pblk + ci * (CHUNK // 128)) * 4
        pltpu.sync_copy(pts_hbm.at[pl.ds(rbase, CHUNK * 4 // 128)], pts_v)

        def pbody(j, carry):
            pids = j * 16 + lane
            rx = jnp.zeros((16,), jnp.int32) + lax.shift_right_logical(j, 3) * 4
            pcol = lax.bitwise_and(j, 7) * 16 + lane
            x = plsc.load_gather(pts_v, [rx, pcol])
            y = plsc.load_gather(pts_v, [rx + 1, pcol])
            z = plsc.load_gather(pts_v, [rx + 2, pcol])
            w = plsc.load_gather(pts_v, [rx + 3, pcol])
            xs = x * 2.0 + 100.0
            ys = y * 2.0 + 100.0
            xi = xs.astype(jnp.int32)
            yi = ys.astype(jnp.int32)
            valid = ((xs >= 0.0) & (xi < W) & (ys >= 0.0) & (yi < H))
            bin_ = jnp.where(valid, yi * W + xi + boff, ACC)
            plsc.store_scatter(rows_v.at[sl], [pids, cc[0]], x)
            plsc.store_scatter(rows_v.at[sl], [pids, cc[1]], y)
            plsc.store_scatter(rows_v.at[sl], [pids, cc[2]], z)
            plsc.store_scatter(rows_v.at[sl], [pids, cc[3]], w)
            flat = j * 16
            plsc.store_scatter(
                idx_v.at[sl], [lax.shift_right_logical(flat + lane, 7),
                               lax.bitwise_and(flat + lane, 127)], bin_)
            return carry
        lax.fori_loop(0, CHUNK // 16, pbody, 0)

        for g in range(NGRP):
            pending[sl].append(pltpu.async_copy(
                rows_v.at[sl, pl.ds(g * GRP, GRP), :],
                acc_sh.at[idx_v.at[sl, g]], sem, add=True))

    for sl in range(2):
        for d in pending[sl]:
            d.wait()
    plsc.subcore_barrier()

    for t in range(2):
        sl = pl.ds(s * RPT + t * ZCH, ZCH)
        pltpu.sync_copy(acc_sh.at[sl], out_hbm.at[c, sl])


def _sc_scatter(points_flat):
    mesh = plsc.VectorSubcoreMesh(core_axis_name="c", subcore_axis_name="s",
                                  num_cores=CORES, num_subcores=SUBS)
    return pl.kernel(
        _sc_body,
        out_type=jax.ShapeDtypeStruct((CORES, ACCPAD, 8), jnp.float32),
        mesh=mesh,
        scratch_types=[
            pltpu.VMEM((CHUNK * 4 // 128, 128), jnp.float32),
            pltpu.VMEM((2, CHUNK, 8), jnp.float32),
            pltpu.VMEM((2, NGRP, GRP), jnp.int32),
            pltpu.VMEM((ZCH, 8), jnp.float32),
            pltpu.VMEM_SHARED((ACCPAD, 8), jnp.float32),
            pltpu.SemaphoreType.DMA,
        ],
        compiler_params=pltpu.CompilerParams(needs_layout_passes=False,
                                             use_tc_tiling_on_sc=False),
    )(points_flat)


_CHUNKS = [(o, min(ROWS_BLK, K - o)) for o in range(0, K, ROWS_BLK)]


def _mlp_ln_body(acc_hbm, w1t_ref, b1_ref, w2t_ref, b2_ref, g_ref, bt_ref,
                 out_ref, abuf, sem):
    b = pl.program_id(0)
    core = b // 2
    rowbase = (b % 2) * K

    def start(ci):
        off, sz = _CHUNKS[ci]
        pltpu.make_async_copy(
            acc_hbm.at[core, pl.ds(rowbase + off, sz), :],
            abuf.at[ci % 2, pl.ds(0, sz), :], sem.at[ci % 2]).start()

    def wait(ci):
        off, sz = _CHUNKS[ci]
        pltpu.make_async_copy(
            acc_hbm.at[core, pl.ds(rowbase + off, sz), :],
            abuf.at[ci % 2, pl.ds(0, sz), :], sem.at[ci % 2]).wait()

    start(0)
    for ci in range(len(_CHUNKS)):
        off, sz = _CHUNKS[ci]
        if ci + 1 < len(_CHUNKS):
            start(ci + 1)
        wait(ci)
        a = abuf[ci % 2, pl.ds(0, sz), :]
        denom = jnp.maximum(a[:, 4:5], 1.0)
        col = jax.lax.broadcasted_iota(jnp.int32, (sz, 8), 1)
        feat = jnp.where(col == 4, a, a / denom)
        h_t = jnp.maximum(
            jax.lax.dot_general(w1t_ref[...], feat, (((1,), (1,)), ((), ())),
                                preferred_element_type=jnp.float32) + b1_ref[...],
            0.0)
        tok_t = jnp.dot(w2t_ref[...], h_t,
                        preferred_element_type=jnp.float32) + b2_ref[...]
        mu = jnp.mean(tok_t, axis=0, keepdims=True)
        var = jnp.mean((tok_t - mu) ** 2, axis=0, keepdims=True)
        out_ref[0, :, pl.ds(off, sz)] = (
            (tok_t - mu) * jax.lax.rsqrt(var + 1e-5) * g_ref[...] + bt_ref[...])


def _mlp_ln(acc, W1t, b1, W2t, b2, gamma, beta):
    grid = (B,)
    return pl.pallas_call(
        _mlp_ln_body,
        grid=grid,
        in_specs=[
            pl.BlockSpec(memory_space=pl.ANY),
            pl.BlockSpec((64, 8), lambda b: (0, 0)),
            pl.BlockSpec((64, 1), lambda b: (0, 0)),
            pl.BlockSpec((EMBED, 64), lambda b: (0, 0)),
            pl.BlockSpec((EMBED, 1), lambda b: (0, 0)),
            pl.BlockSpec((EMBED, 1), lambda b: (0, 0)),
            pl.BlockSpec((EMBED, 1), lambda b: (0, 0)),
        ],
        out_specs=pl.BlockSpec((1, EMBED, K), lambda b: (b, 0, 0)),
        out_shape=jax.ShapeDtypeStruct((B, EMBED, K), jnp.float32),
        scratch_shapes=[
            pltpu.VMEM((2, ROWS_BLK, 8), jnp.float32),
            pltpu.SemaphoreType.DMA((2,)),
        ],
    )(acc, W1t, b1, W2t, b2, gamma, beta)


def kernel(points, W1, b1, W2, b2, gamma, beta):
    pts_lin = (points.reshape(B, N // 128, 128, 4)
               .swapaxes(2, 3)
               .reshape(B * N * 4 // 128, 128))
    acc = _sc_scatter(pts_lin)
    W1t = jnp.zeros((64, 8), jnp.float32).at[:, :5].set(W1.T)
    out = _mlp_ln(acc, W1t, b1.reshape(64, 1), W2.T, b2.reshape(EMBED, 1),
                  gamma.reshape(EMBED, 1), beta.reshape(EMBED, 1))
    return out.swapaxes(1, 2)
